# baseline (speedup 1.0000x reference)
import functools

import numpy as np
import jax
import jax.numpy as jnp
from jax import lax
from jax.experimental import pallas as pl
from jax.experimental.pallas import tpu as pltpu
from jax.experimental.pallas import tpu_sc as plsc

_CONTEXT_WINDOW = 200
_EMBEDDING_DIM = 64
_LANES = 16
_BLK = 128


def _pe_np(context_window, embedding_dim):
    pos = np.arange(context_window, dtype=np.float32)[:, None]
    i = np.arange(embedding_dim, dtype=np.float32)[None, :]
    angle = pos / np.power(10000.0, i / embedding_dim)
    pe = np.where((np.arange(embedding_dim)[None, :] % 2) == 0,
                  np.sin(angle), np.cos(angle))
    return pe.astype(np.float32)


@functools.lru_cache(maxsize=None)
def _build_sc_call(B, D, b_per_w):
    mesh = plsc.VectorSubcoreMesh(core_axis_name="c", subcore_axis_name="s")
    info = plsc.get_sparse_core_info()
    nc = info.num_cores
    n_active = B // b_per_w

    @functools.partial(
        pl.kernel,
        mesh=mesh,
        out_type=jax.ShapeDtypeStruct((B * D,), jnp.float32),
        scratch_types=[
            pltpu.VMEM((2 * _LANES,), jnp.int32),
            pltpu.VMEM((b_per_w, D, _BLK), jnp.float32),
            pltpu.VMEM((b_per_w * D,), jnp.float32),
            pltpu.VMEM((b_per_w * D,), jnp.float32),
            pltpu.SemaphoreType.DMA((b_per_w,)),
            pltpu.SemaphoreType.DMA,
        ],
        compiler_params=pltpu.CompilerParams(needs_layout_passes=False),
    )
    def sc_embed(x_hbm, tab_t_hbm, pe_hbm, out_hbm, idx_v, blocks_v, rows_v,
                 pe_v, sem, sem_pe):
        wid = lax.axis_index("s") * nc + lax.axis_index("c")

        @pl.when(wid < n_active)
        def _():
            base = wid * b_per_w
            pe_cp = pltpu.async_copy(
                pe_hbm.at[pl.ds(base * D, b_per_w * D)], pe_v, sem_pe)
            pltpu.sync_copy(x_hbm.at[pl.ds(base, b_per_w)],
                            idx_v.at[pl.ds(0, b_per_w)])

            def _row_at(i):
                return idx_v[pl.ds(i, _LANES)][0]

            for i in range(b_per_w):
                row = _row_at(i)
                col = row & (_BLK - 1)
                blk = pl.multiple_of(row - col, _BLK)
                pltpu.async_copy(
                    tab_t_hbm.at[:, pl.ds(blk, _BLK)], blocks_v.at[i],
                    sem.at[i])
            pe_cp.wait()
            lane = lax.iota(jnp.int32, _LANES)

            for i in range(b_per_w):
                pltpu.make_async_copy(
                    tab_t_hbm.at[:, pl.ds(0, _BLK)], blocks_v.at[i], sem.at[i]
                ).wait()
                col_b = jnp.full((_LANES,), _row_at(i) & (_BLK - 1), jnp.int32)
                sel_i = jnp.full((_LANES,), i, jnp.int32)

                def chunk(j, c2, i=i, col_b=col_b, sel_i=sel_i):
                    s = pl.ds(i * D + j * _LANES, _LANES)
                    val = plsc.load_gather(
                        blocks_v, [sel_i, j * _LANES + lane, col_b])
                    rows_v[s] = val + pe_v[s]
                    return c2

                lax.fori_loop(0, D // _LANES, chunk, 0)
            pltpu.sync_copy(rows_v, out_hbm.at[pl.ds(base * D, b_per_w * D)])

    return sc_embed


def kernel(x, table):
    pe = _pe_np(_CONTEXT_WINDOW, _EMBEDDING_DIM).reshape(-1)
    out = _build_sc_call(_CONTEXT_WINDOW, _EMBEDDING_DIM, 8)(
        x, table.T, jnp.asarray(pe))
    return out.reshape(_CONTEXT_WINDOW, _EMBEDDING_DIM)


# speedup vs baseline: 1.0274x; 1.0274x over previous
import functools

import numpy as np
import jax
import jax.numpy as jnp
from jax import lax
from jax.experimental import pallas as pl
from jax.experimental.pallas import tpu as pltpu
from jax.experimental.pallas import tpu_sc as plsc

_CONTEXT_WINDOW = 200
_EMBEDDING_DIM = 64
_LANES = 16
_BLK = 128


def _pe_np(context_window, embedding_dim):
    pos = np.arange(context_window, dtype=np.float32)[:, None]
    i = np.arange(embedding_dim, dtype=np.float32)[None, :]
    angle = pos / np.power(10000.0, i / embedding_dim)
    pe = np.where((np.arange(embedding_dim)[None, :] % 2) == 0,
                  np.sin(angle), np.cos(angle))
    return pe.astype(np.float32)


@functools.lru_cache(maxsize=None)
def _build_sc_call(B, D, b_per_w):
    mesh = plsc.VectorSubcoreMesh(core_axis_name="c", subcore_axis_name="s")
    info = plsc.get_sparse_core_info()
    nc = info.num_cores
    n_active = B // b_per_w

    @functools.partial(
        pl.kernel,
        mesh=mesh,
        out_type=jax.ShapeDtypeStruct((B * D,), jnp.float32),
        scratch_types=[
            pltpu.VMEM((2 * _LANES,), jnp.int32),
            pltpu.VMEM((b_per_w, D, _BLK), jnp.float32),
            pltpu.VMEM((b_per_w * D,), jnp.float32),
            pltpu.VMEM((b_per_w * D,), jnp.float32),
            pltpu.SemaphoreType.DMA,
            pltpu.SemaphoreType.DMA,
        ],
        compiler_params=pltpu.CompilerParams(needs_layout_passes=False),
    )
    def sc_embed(x_hbm, tab_t_hbm, pe_hbm, out_hbm, idx_v, blocks_v, rows_v,
                 pe_v, sem, sem_pe):
        wid = lax.axis_index("s") * nc + lax.axis_index("c")

        @pl.when(wid < n_active)
        def _():
            base = wid * b_per_w
            pe_cp = pltpu.async_copy(
                pe_hbm.at[pl.ds(base * D, b_per_w * D)], pe_v, sem_pe)
            pltpu.sync_copy(x_hbm.at[pl.ds(base, b_per_w)],
                            idx_v.at[pl.ds(0, b_per_w)])

            def _row_at(i):
                return idx_v[pl.ds(i, _LANES)][0]

            def issue(i, carry):
                row = _row_at(i)
                col = row & (_BLK - 1)
                blk = pl.multiple_of(row - col, _BLK)
                pltpu.async_copy(
                    tab_t_hbm.at[:, pl.ds(blk, _BLK)], blocks_v.at[i], sem)
                return carry

            lax.fori_loop(0, b_per_w, issue, 0)
            pe_cp.wait()

            def drain(i, carry):
                pltpu.make_async_copy(
                    tab_t_hbm.at[:, pl.ds(0, _BLK)], blocks_v.at[i], sem
                ).wait()
                return carry

            lax.fori_loop(0, b_per_w, drain, 0)
            lane = lax.iota(jnp.int32, _LANES)

            def select(i, carry):
                col_b = jnp.full((_LANES,), _row_at(i) & (_BLK - 1), jnp.int32)
                sel_i = jnp.full((_LANES,), i, jnp.int32)

                def chunk(j, c2):
                    s = pl.ds(i * D + j * _LANES, _LANES)
                    val = plsc.load_gather(
                        blocks_v, [sel_i, j * _LANES + lane, col_b])
                    rows_v[s] = val + pe_v[s]
                    return c2

                lax.fori_loop(0, D // _LANES, chunk, 0)
                return carry

            lax.fori_loop(0, b_per_w, select, 0)
            pltpu.sync_copy(rows_v, out_hbm.at[pl.ds(base * D, b_per_w * D)])

    return sc_embed


def kernel(x, table):
    pe = _pe_np(_CONTEXT_WINDOW, _EMBEDDING_DIM).reshape(-1)
    out = _build_sc_call(_CONTEXT_WINDOW, _EMBEDDING_DIM, 8)(
        x, table.T, jnp.asarray(pe))
    return out.reshape(_CONTEXT_WINDOW, _EMBEDDING_DIM)
